# baseline (speedup 1.0000x reference)
import jax
import jax.numpy as jnp
from jax.experimental import pallas as pl

BATCH_BLOCK = 128


def _body(x_ref, pos_ref, out_ref):
    out_ref[...] = x_ref[...] + pos_ref[...][None, :, :]


def kernel(x, pos_table):
    batch, seq, dim = x.shape
    grid = (batch // BATCH_BLOCK,)
    return pl.pallas_call(
        _body,
        grid=grid,
        in_specs=[
            pl.BlockSpec((BATCH_BLOCK, seq, dim), lambda i: (i, 0, 0)),
            pl.BlockSpec((seq, dim), lambda i: (0, 0)),
        ],
        out_specs=pl.BlockSpec((BATCH_BLOCK, seq, dim), lambda i: (i, 0, 0)),
        out_shape=jax.ShapeDtypeStruct((batch, seq, dim), x.dtype),
    )(x, pos_table)


# speedup vs baseline: 5.8945x; 5.8945x over previous
import jax
import jax.numpy as jnp
from jax.experimental import pallas as pl

ROW_BLOCK = 512
LANES = 128


def _body(x_ref, pos_ref, out_ref):
    pv = pos_ref[...]
    for g in range(x_ref.shape[1] // LANES):
        sl = pl.ds(g * LANES, LANES)
        out_ref[:, sl] = x_ref[:, sl] + pv


def kernel(x, pos_table):
    b, s, d = x.shape
    k = s * d
    xt = jnp.transpose(x, (1, 2, 0)).reshape(k, b)
    posb = jnp.broadcast_to(pos_table.reshape(k, 1), (k, LANES))
    out = pl.pallas_call(
        _body,
        grid=(k // ROW_BLOCK,),
        in_specs=[
            pl.BlockSpec((ROW_BLOCK, b), lambda i: (i, 0)),
            pl.BlockSpec((ROW_BLOCK, LANES), lambda i: (i, 0)),
        ],
        out_specs=pl.BlockSpec((ROW_BLOCK, b), lambda i: (i, 0)),
        out_shape=jax.ShapeDtypeStruct((k, b), x.dtype),
    )(xt, posb)
    return jnp.transpose(out.reshape(s, d, b), (2, 0, 1))
